# Initial kernel scaffold; baseline (speedup 1.0000x reference)
import jax
import jax.numpy as jnp
from jax.experimental import pallas as pl


def kernel(x, edge_index, num_nodes):
    raise NotImplementedError("write your pallas kernel here")


# speedup vs baseline: 4.0618x; 4.0618x over previous
import functools

import jax
import jax.numpy as jnp
from jax import lax
from jax.experimental import pallas as pl
from jax.experimental.pallas import tpu as pltpu
from jax.experimental.pallas import tpu_sc as plsc

N_NODES = 10000
N_EDGES = 320000
D_FEAT = 128
MAX_HOP = 3

NC = 2
NS = 16
N_PAD = 10240
NODES_PER_CORE = N_PAD // NC
ROWS_PER_TILE = NODES_PER_CORE // NS
DUMP_ROW = NODES_PER_CORE
ACC_ROWS = NODES_PER_CORE + 8
EDGES_PER_TILE = N_EDGES // NS
CHUNK = 80
CHUNKS_PER_TILE = EDGES_PER_TILE // CHUNK


def _sc_body(x, src_idx, dst_idx, zeros, out, acc, rows, srci, dsti, sem):
    c = lax.axis_index("c")
    s = lax.axis_index("s")

    pltpu.sync_copy(src_idx.at[c, s], srci)
    pltpu.sync_copy(dst_idx.at[s], dsti)

    row0 = s * ROWS_PER_TILE
    out_row0 = c * NODES_PER_CORE + s * ROWS_PER_TILE

    for h in range(MAX_HOP):
        pltpu.sync_copy(zeros, acc.at[pl.ds(row0, ROWS_PER_TILE)])
        plsc.subcore_barrier()

        table = x if h == 0 else out.at[h - 1]

        def chunk_body(k, _):
            pltpu.async_copy(table.at[dsti.at[k]], rows, sem).wait()
            pltpu.sync_copy(rows, acc.at[srci.at[k]], add=True)
            return ()

        lax.fori_loop(0, CHUNKS_PER_TILE, chunk_body, ())

        plsc.subcore_barrier()
        pltpu.sync_copy(acc.at[pl.ds(row0, ROWS_PER_TILE)],
                        out.at[h, pl.ds(out_row0, ROWS_PER_TILE)])


@functools.partial(
    pl.kernel,
    out_type=jax.ShapeDtypeStruct((MAX_HOP, N_PAD, D_FEAT), jnp.float32),
    mesh=plsc.VectorSubcoreMesh(core_axis_name="c", subcore_axis_name="s"),
    scratch_types=[
        pltpu.VMEM_SHARED((ACC_ROWS, D_FEAT), jnp.float32),
        pltpu.VMEM((CHUNK, D_FEAT), jnp.float32),
        pltpu.VMEM((CHUNKS_PER_TILE, CHUNK), jnp.int32),
        pltpu.VMEM((CHUNKS_PER_TILE, CHUNK), jnp.int32),
        pltpu.SemaphoreType.DMA,
    ],
)
def _hops_kernel(x, src_idx, dst_idx, zeros, out, acc, rows, srci, dsti, sem):
    _sc_body(x, src_idx, dst_idx, zeros, out, acc, rows, srci, dsti, sem)


def kernel(x, edge_index, num_nodes):
    del num_nodes
    src = edge_index[0]
    dst = edge_index[1]
    core = src // NODES_PER_CORE
    local = src - core * NODES_PER_CORE
    srcm = jnp.stack(
        [jnp.where(core == c, local, DUMP_ROW) for c in range(NC)]
    ).reshape(NC, NS, CHUNKS_PER_TILE, CHUNK)
    dst3 = dst.reshape(NS, CHUNKS_PER_TILE, CHUNK)
    zeros = jnp.zeros((ROWS_PER_TILE, D_FEAT), jnp.float32)
    y = _hops_kernel(x, srcm, dst3, zeros)[:, :N_NODES]
    return jnp.concatenate([x[:, None], jnp.transpose(y, (1, 0, 2))], axis=1)
